# Initial kernel scaffold; baseline (speedup 1.0000x reference)
import jax
import jax.numpy as jnp
from jax.experimental import pallas as pl


def kernel(x, edge_index, edge_weight, W, b, gamma, beta):
    raise NotImplementedError("write your pallas kernel here")


# speedup vs baseline: 11.6732x; 11.6732x over previous
import functools

import jax
import jax.numpy as jnp
from jax import lax
from jax.experimental import pallas as pl
from jax.experimental.pallas import tpu as pltpu
from jax.experimental.pallas import tpu_sc as plsc

B, C, N, E, K = 4, 128, 10000, 320000, 3
NB = B * N
EB = 128
EPAD = 323584
NPAD = 10240
EPT = EPAD // 16
EPT_B = EPT // EB
EPH = EPAD // 32
EPH_B = EPH // EB
NSL = N // 16
F32 = jnp.float32
I32 = jnp.int32

_mesh = plsc.VectorSubcoreMesh(core_axis_name="c", subcore_axis_name="s")
_sc_params = pltpu.CompilerParams(needs_layout_passes=False,
                                  use_tc_tiling_on_sc=False)


def _rsqrt16(d):
    i = plsc.bitcast(d, I32)
    i = jnp.full((16,), 0x5F3759DF, I32) - lax.shift_right_arithmetic(i, 1)
    y = plsc.bitcast(i, F32)
    for _ in range(4):
        y = y * (1.5 - 0.5 * d * y * y)
    return y


def _norm_body(row_h, col_h, ew_h, lw_h,
               pdeg2, pdeg1, rbuf, cbuf, wbuf, tmp, accb, dv, dvfull, lwbuf,
               sdeg, sdinv):
    c = lax.axis_index("c")
    s = lax.axis_index("s")
    lanes = lax.iota(I32, 16)
    rows8 = lanes & 7
    m_lo = lanes < 8
    m_hi = lanes >= 8
    z16 = jnp.zeros((16,), F32)

    def zero_pdeg(i, _):
        for r in range(8):
            pdeg2[pl.ds(r * NPAD + i * 16, 16)] = z16
        return 0
    lax.fori_loop(0, NPAD // 16, zero_pdeg, 0)

    def deg_batch(k, _):
        eoff = s * EPT + k * EB
        pltpu.sync_copy(row_h.at[pl.ds(eoff, EB)], rbuf)
        pltpu.sync_copy(col_h.at[pl.ds(eoff, EB)], cbuf)
        pltpu.sync_copy(ew_h.at[pl.ds(eoff, EB)], wbuf)
        for j in range(8):
            r16 = rbuf[pl.ds(j * 16, 16)]
            c16 = cbuf[pl.ds(j * 16, 16)]
            w16 = wbuf[pl.ds(j * 16, 16)]
            w16 = jnp.where(r16 == c16, 0.0, w16)
            fidx = rows8 * NPAD + r16
            plsc.addupdate_scatter(pdeg2, [fidx], w16, mask=m_lo)
            plsc.addupdate_scatter(pdeg2, [fidx], w16, mask=m_hi)
        return 0
    lax.fori_loop(0, EPT_B, deg_batch, 0)

    def red8(i, _):
        v = pdeg2[pl.ds(i * 16, 16)]
        for r in range(1, 8):
            v = v + pdeg2[pl.ds(r * NPAD + i * 16, 16)]
        pdeg1[pl.ds(i * 16, 16)] = v
        return 0
    lax.fori_loop(0, NPAD // 16, red8, 0)
    pltpu.sync_copy(pdeg1, sdeg.at[s])
    plsc.subcore_barrier()

    def zacc(i, _):
        accb[pl.ds(i * 16, 16)] = z16
        return 0
    lax.fori_loop(0, 40, zacc, 0)
    for j in range(16):
        pltpu.sync_copy(sdeg.at[j, pl.ds(s * 640, 640)], tmp)
        def addt(i, _):
            accb[pl.ds(i * 16, 16)] = accb[pl.ds(i * 16, 16)] + tmp[pl.ds(i * 16, 16)]
            return 0
        lax.fori_loop(0, 40, addt, 0)

    def dinv_chunk(i, _):
        d = accb[pl.ds(i * 16, 16)]
        y = _rsqrt16(d)
        dv[pl.ds(i * 16, 16)] = jnp.where(d > 0.0, y, 0.0)
        return 0
    lax.fori_loop(0, 40, dinv_chunk, 0)
    pltpu.sync_copy(dv, sdinv.at[pl.ds(s * 640, 640)])
    plsc.subcore_barrier()
    pltpu.sync_copy(sdinv, dvfull)

    base2 = c * (EPAD // 2) + s * EPH

    def lw_batch(k, _):
        eoff = base2 + k * EB
        pltpu.sync_copy(row_h.at[pl.ds(eoff, EB)], rbuf)
        pltpu.sync_copy(col_h.at[pl.ds(eoff, EB)], cbuf)
        pltpu.sync_copy(ew_h.at[pl.ds(eoff, EB)], wbuf)
        for j in range(8):
            r16 = rbuf[pl.ds(j * 16, 16)]
            c16 = cbuf[pl.ds(j * 16, 16)]
            w16 = wbuf[pl.ds(j * 16, 16)]
            w16 = jnp.where(r16 == c16, 0.0, w16)
            dr = plsc.load_gather(dvfull, [r16])
            dc = plsc.load_gather(dvfull, [c16])
            lwbuf[pl.ds(j * 16, 16)] = -(dr * w16 * dc)
        pltpu.sync_copy(lwbuf, lw_h.at[pl.ds(eoff, EB)])
        return 0
    lax.fori_loop(0, EPH_B, lw_batch, 0)


_norm_call = functools.partial(
    pl.kernel,
    out_type=jax.ShapeDtypeStruct((EPAD,), F32),
    mesh=_mesh,
    scratch_types=[
        pltpu.VMEM((8 * NPAD,), F32),
        pltpu.VMEM((NPAD,), F32),
        pltpu.VMEM((EB,), I32),
        pltpu.VMEM((EB,), I32),
        pltpu.VMEM((EB,), F32),
        pltpu.VMEM((640,), F32),
        pltpu.VMEM((640,), F32),
        pltpu.VMEM((640,), F32),
        pltpu.VMEM((NPAD,), F32),
        pltpu.VMEM((EB,), F32),
        pltpu.VMEM_SHARED((16, NPAD), F32),
        pltpu.VMEM_SHARED((NPAD,), F32),
    ],
    compiler_params=_sc_params,
)(_norm_body)


def _prop_body(t_h, row_h, col_h, lw_h, out_h,
               acc, rraw, ridx, cidx, lwb, gbuf, zbuf, sbuf, gsem):
    c = lax.axis_index("c")
    s = lax.axis_index("s")
    z16 = jnp.zeros((16,), F32)

    def zero_zbuf(i, _):
        for j in range(8):
            zbuf[i, pl.ds(j * 16, 16)] = z16
        return 0
    lax.fori_loop(0, EB, zero_zbuf, 0)

    for bi in range(2):
        b = c * 2 + bi
        bN = b * N
        for q in range(5):
            pltpu.sync_copy(zbuf.at[pl.ds(0, 125)],
                            acc.at[pl.ds(s * NSL + q * 125, 125)])
        plsc.subcore_barrier()

        def ebatch(k, _):
            eoff = s * EPT + k * EB
            pltpu.sync_copy(row_h.at[pl.ds(eoff, EB)], rraw)
            pltpu.sync_copy(col_h.at[pl.ds(eoff, EB)], cidx)
            pltpu.sync_copy(lw_h.at[pl.ds(eoff, EB)], lwb)
            for j in range(8):
                ridx[pl.ds(j * 16, 16)] = rraw[pl.ds(j * 16, 16)] + bN
            pltpu.async_copy(t_h.at[ridx], gbuf, gsem).wait()

            def scale_grp(g, _):
                lw16 = lwb[pl.ds(g * 16, 16)]
                for i in range(16):
                    w = lw16[i]
                    e = g * 16 + i
                    for j in range(8):
                        gbuf[e, pl.ds(j * 16, 16)] = gbuf[e, pl.ds(j * 16, 16)] * w
                return 0
            lax.fori_loop(0, EB // 16, scale_grp, 0)
            pltpu.sync_copy(gbuf, acc.at[cidx], add=True)
            return 0
        lax.fori_loop(0, EPT_B, ebatch, 0)
        plsc.subcore_barrier()

        for q in range(5):
            ro = s * NSL + q * 125
            pltpu.sync_copy(acc.at[pl.ds(ro, 125)], sbuf.at[pl.ds(0, 125)])
            pltpu.sync_copy(sbuf.at[pl.ds(0, 125)], out_h.at[pl.ds(bN + ro, 125)])
        if bi == 0:
            plsc.subcore_barrier()


_prop_call = functools.partial(
    pl.kernel,
    out_type=jax.ShapeDtypeStruct((NB, C), F32),
    mesh=_mesh,
    scratch_types=[
        pltpu.VMEM_SHARED((N, C), F32),
        pltpu.VMEM((EB,), I32),
        pltpu.VMEM((EB,), I32),
        pltpu.VMEM((EB,), I32),
        pltpu.VMEM((EB,), F32),
        pltpu.VMEM((EB, C), F32),
        pltpu.VMEM((EB, C), F32),
        pltpu.VMEM((EB, C), F32),
        pltpu.SemaphoreType.DMA,
    ],
    compiler_params=_sc_params,
)(_prop_body)


TN = 2000


def _mm_body(x_ref, t1_ref, t2_ref, w_ref, bb_ref, y_ref, s_ref):
    kidx = pl.program_id(0)
    u = jnp.concatenate([x_ref[...], t1_ref[...], 2.0 * t2_ref[...]], axis=1)
    wc = jnp.concatenate([w_ref[0] - w_ref[2], w_ref[1], w_ref[2]], axis=0)
    yb = jnp.dot(u, wc, preferred_element_type=F32) + bb_ref[0:1, :]
    y_ref[...] = yb

    @pl.when(kidx == 0)
    def _():
        s_ref[...] = jnp.zeros((8, C), F32)
    s_ref[0:1, :] += jnp.sum(yb, axis=0, keepdims=True)
    s_ref[1:2, :] += jnp.sum(yb * yb, axis=0, keepdims=True)


def _bn_body(y_ref, sc_ref, sh_ref, o_ref):
    o_ref[...] = jnp.maximum(y_ref[...] * sc_ref[0:1, :] + sh_ref[0:1, :], 0.0)


def kernel(x, edge_index, edge_weight, W, b, gamma, beta):
    tflat = jnp.transpose(x, (0, 2, 1)).reshape(NB, C)
    pad = EPAD - E
    row_p = jnp.concatenate([edge_index[0], jnp.zeros((pad,), I32)])
    col_p = jnp.concatenate([edge_index[1], jnp.zeros((pad,), I32)])
    ew_p = jnp.concatenate([edge_weight, jnp.zeros((pad,), F32)])

    lw = _norm_call(row_p, col_p, ew_p)
    t1 = _prop_call(tflat, row_p, col_p, lw)
    t2 = _prop_call(t1, row_p, col_p, lw)

    bb = jnp.broadcast_to(b[None, :], (8, C))
    y, sums = pl.pallas_call(
        _mm_body,
        grid=(NB // TN,),
        in_specs=[
            pl.BlockSpec((TN, C), lambda k: (k, 0)),
            pl.BlockSpec((TN, C), lambda k: (k, 0)),
            pl.BlockSpec((TN, C), lambda k: (k, 0)),
            pl.BlockSpec((K, C, C), lambda k: (0, 0, 0)),
            pl.BlockSpec((8, C), lambda k: (0, 0)),
        ],
        out_specs=[
            pl.BlockSpec((TN, C), lambda k: (k, 0)),
            pl.BlockSpec((8, C), lambda k: (0, 0)),
        ],
        out_shape=[
            jax.ShapeDtypeStruct((NB, C), F32),
            jax.ShapeDtypeStruct((8, C), F32),
        ],
    )(tflat, t1, t2, W, bb)

    mean = sums[0] / NB
    var = sums[1] / NB - mean * mean
    rstd = lax.rsqrt(var + 1e-5)
    scale = gamma * rstd
    shift = beta - mean * scale
    scb = jnp.broadcast_to(scale[None, :], (8, C))
    shb = jnp.broadcast_to(shift[None, :], (8, C))

    out = pl.pallas_call(
        _bn_body,
        grid=(NB // TN,),
        in_specs=[
            pl.BlockSpec((TN, C), lambda k: (k, 0)),
            pl.BlockSpec((8, C), lambda k: (0, 0)),
            pl.BlockSpec((8, C), lambda k: (0, 0)),
        ],
        out_specs=pl.BlockSpec((TN, C), lambda k: (k, 0)),
        out_shape=jax.ShapeDtypeStruct((NB, C), F32),
    )(y, scb, shb)

    return out.reshape(B, N, C).transpose(0, 2, 1)
